# Initial kernel scaffold; baseline (speedup 1.0000x reference)
import jax
import jax.numpy as jnp
from jax.experimental import pallas as pl


def kernel(seq1, seq2, adj, sparse, msk, samp_bias1, samp_bias2, W, b_gcn, prelu_a, M, disc_bias):
    raise NotImplementedError("write your pallas kernel here")


# speedup vs baseline: 6.1793x; 6.1793x over previous
import functools

import jax
import jax.numpy as jnp
from jax import lax
from jax.experimental import pallas as pl
from jax.experimental.pallas import tpu as pltpu
from jax.experimental.pallas import tpu_sc as plsc

N = 10000
E = 320000
FT = 128
HID = 128
NS = 16
CK = 128
NCHT = E // CK
CPS = NCHT // NS
XTRA = NCHT - CPS * NS
RPS = 624
TAIL = N - NS * RPS


def _mm_body(s_ref, w_ref, o_ref):
    o_ref[0] = jnp.dot(s_ref[0], w_ref[...], preferred_element_type=jnp.float32)


def _mm(seqs, W):
    BR = 2000
    return pl.pallas_call(
        _mm_body,
        grid=(2, N // BR),
        in_specs=[pl.BlockSpec((1, BR, FT), lambda g, i: (g, i, 0)),
                  pl.BlockSpec((FT, HID), lambda g, i: (0, 0))],
        out_specs=pl.BlockSpec((1, BR, HID), lambda g, i: (g, i, 0)),
        out_shape=jax.ShapeDtypeStruct((2, N, HID), jnp.float32),
    )(seqs, W)


def _sc_agg(fts2, srcx, dst, zrows):
    @functools.partial(
        pl.kernel,
        out_type=jax.ShapeDtypeStruct((2 * N, HID), jnp.float32),
        mesh=plsc.VectorSubcoreMesh(core_axis_name="c", subcore_axis_name="s"),
        scratch_types=[
            pltpu.VMEM((CK,), jnp.int32),
            pltpu.VMEM((CK,), jnp.int32),
            pltpu.VMEM((CK, HID), jnp.float32),
            pltpu.VMEM_SHARED((N, HID), jnp.float32),
            pltpu.SemaphoreType.DMA,
        ],
    )
    def k(fts_h, srcx_h, dst_h, z_h, out_h, src_v, dst_v, rows_v, acc, sem):
        cid = lax.axis_index("c")
        sid = lax.axis_index("s")
        pltpu.sync_copy(z_h, acc.at[pl.ds(sid * RPS, RPS)])

        @pl.when(sid == 0)
        def _():
            pltpu.sync_copy(z_h.at[pl.ds(0, TAIL)],
                            acc.at[pl.ds(NS * RPS, TAIL)])

        plsc.subcore_barrier()

        def do_chunk(q):
            pltpu.sync_copy(srcx_h.at[pl.ds(cid * E + q * CK, CK)], src_v)
            pltpu.sync_copy(dst_h.at[pl.ds(q * CK, CK)], dst_v)
            pltpu.async_copy(fts_h.at[src_v], rows_v, sem).wait()
            pltpu.sync_copy(rows_v, acc.at[dst_v], add=True)

        def body(j, carry):
            do_chunk(sid * CPS + j)
            return carry

        lax.fori_loop(0, CPS, body, 0)

        @pl.when(sid < XTRA)
        def _():
            do_chunk(NS * CPS + sid)

        plsc.subcore_barrier()
        pltpu.sync_copy(acc.at[pl.ds(sid * RPS, RPS)],
                        out_h.at[pl.ds(cid * N + sid * RPS, RPS)])

        @pl.when(sid == 0)
        def _():
            pltpu.sync_copy(acc.at[pl.ds(NS * RPS, TAIL)],
                            out_h.at[pl.ds(cid * N + NS * RPS, TAIL)])

    return k(fts2, srcx, dst, zrows)


def _post_body(agg_ref, b_ref, a_ref, m_ref, sb_ref, h_ref, ret_ref, cm_ref):
    g = pl.program_id(0)
    x = agg_ref[0] + b_ref[...]
    a = a_ref[0, 0]
    h = jnp.where(x > 0.0, x, a * x)

    @pl.when(g == 0)
    def _():
        h_ref[0] = h
        c = jax.nn.sigmoid(jnp.sum(h, axis=0, keepdims=True) * (1.0 / N))
        cm_ref[...] = jnp.dot(c, m_ref[...], preferred_element_type=jnp.float32)

    s = jnp.sum(h * cm_ref[...], axis=-1)
    ret_ref[...] = s.reshape(1, 1, N) + sb_ref[...]


def _post(agg, b2, a2, M, sb):
    return pl.pallas_call(
        _post_body,
        grid=(2,),
        in_specs=[pl.BlockSpec((1, N, HID), lambda g: (g, 0, 0)),
                  pl.BlockSpec((1, HID), lambda g: (0, 0)),
                  pl.BlockSpec((1, 1), lambda g: (0, 0)),
                  pl.BlockSpec((HID, HID), lambda g: (0, 0)),
                  pl.BlockSpec((1, 1, N), lambda g: (g, 0, 0))],
        out_specs=[pl.BlockSpec((1, N, HID), lambda g: (0, 0, 0)),
                   pl.BlockSpec((1, 1, N), lambda g: (g, 0, 0))],
        out_shape=[jax.ShapeDtypeStruct((1, N, HID), jnp.float32),
                   jax.ShapeDtypeStruct((2, 1, N), jnp.float32)],
        scratch_shapes=[pltpu.VMEM((1, HID), jnp.float32)],
    )(agg, b2, a2, M, sb)


def kernel(seq1, seq2, adj, sparse, msk, samp_bias1, samp_bias2, W, b_gcn,
           prelu_a, M, disc_bias):
    seqs = jnp.concatenate([seq1, seq2], axis=0)
    fts2 = _mm(seqs, W).reshape(2 * N, HID)
    src = adj[0]
    dst = adj[1]
    srcx = jnp.concatenate([src, src + N], axis=0)
    zrows = jnp.zeros((RPS, HID), jnp.float32)
    agg = _sc_agg(fts2, srcx, dst, zrows).reshape(2, N, HID)
    b2 = b_gcn.reshape(1, HID)
    a2 = prelu_a.reshape(1, 1)
    sb = (jnp.stack([samp_bias1[0], samp_bias2[0]])
          .reshape(2, 1, N) + disc_bias)
    h1, ret2 = _post(agg, b2, a2, M, sb)
    ret = ret2.reshape(1, 2 * N)
    return (ret, h1)


# speedup vs baseline: 9.5220x; 1.5409x over previous
import functools

import jax
import jax.numpy as jnp
from jax import lax
from jax.experimental import pallas as pl
from jax.experimental.pallas import tpu as pltpu
from jax.experimental.pallas import tpu_sc as plsc

N = 10000
E = 320000
FT = 128
HID = 128
NS = 16
CK = 128
NCHT = E // CK
CPS = NCHT // NS
XTRA = NCHT - CPS * NS
RPS = 624
TAIL = N - NS * RPS


def _mm_body(s_ref, w_ref, o_ref):
    o_ref[0] = jnp.dot(s_ref[0], w_ref[...], preferred_element_type=jnp.float32)


def _mm(seqs, W):
    BR = 2000
    return pl.pallas_call(
        _mm_body,
        grid=(2, N // BR),
        in_specs=[pl.BlockSpec((1, BR, FT), lambda g, i: (g, i, 0)),
                  pl.BlockSpec((FT, HID), lambda g, i: (0, 0))],
        out_specs=pl.BlockSpec((1, BR, HID), lambda g, i: (g, i, 0)),
        out_shape=jax.ShapeDtypeStruct((2, N, HID), jnp.float32),
    )(seqs, W)


def _sc_agg(fts2, srcx, dst, zrows):
    @functools.partial(
        pl.kernel,
        out_type=jax.ShapeDtypeStruct((2 * N, HID), jnp.float32),
        mesh=plsc.VectorSubcoreMesh(core_axis_name="c", subcore_axis_name="s"),
        scratch_types=[
            pltpu.VMEM((CK,), jnp.int32),
            pltpu.VMEM((CK,), jnp.int32),
            pltpu.VMEM((CK,), jnp.int32),
            pltpu.VMEM((CK,), jnp.int32),
            pltpu.VMEM((CK, HID), jnp.float32),
            pltpu.VMEM((CK, HID), jnp.float32),
            pltpu.VMEM_SHARED((N, HID), jnp.float32),
            pltpu.SemaphoreType.DMA,
            pltpu.SemaphoreType.DMA,
        ],
    )
    def k(fts_h, srcx_h, dst_h, z_h, out_h, src0, src1, dst0, dst1,
          rows0, rows1, acc, sem0, sem1):
        cid = lax.axis_index("c")
        sid = lax.axis_index("s")
        pltpu.sync_copy(z_h, acc.at[pl.ds(sid * RPS, RPS)])

        @pl.when(sid == 0)
        def _():
            pltpu.sync_copy(z_h.at[pl.ds(0, TAIL)],
                            acc.at[pl.ds(NS * RPS, TAIL)])

        plsc.subcore_barrier()

        base = sid * CPS

        def load_idx(q, sv, dv):
            pltpu.sync_copy(srcx_h.at[pl.ds(cid * E + q * CK, CK)], sv)
            pltpu.sync_copy(dst_h.at[pl.ds(q * CK, CK)], dv)

        def issue(sv, rv, sem):
            pltpu.async_copy(fts_h.at[sv], rv, sem)

        def wait(sv, rv, sem):
            pltpu.make_async_copy(fts_h.at[sv], rv, sem).wait()

        def scat(rv, dv):
            pltpu.sync_copy(rv, acc.at[dv], add=True)

        load_idx(base + 0, src0, dst0)
        issue(src0, rows0, sem0)
        load_idx(base + 1, src1, dst1)
        issue(src1, rows1, sem1)
        HALF = CPS // 2

        def body(k2, carry):
            wait(src0, rows0, sem0)
            scat(rows0, dst0)

            @pl.when(k2 < HALF - 1)
            def _():
                load_idx(base + 2 * k2 + 2, src0, dst0)
                issue(src0, rows0, sem0)

            wait(src1, rows1, sem1)
            scat(rows1, dst1)

            @pl.when(k2 < HALF - 1)
            def _():
                load_idx(base + 2 * k2 + 3, src1, dst1)
                issue(src1, rows1, sem1)

            return carry

        lax.fori_loop(0, HALF, body, 0)

        @pl.when(sid < XTRA)
        def _():
            q = NS * CPS + sid
            load_idx(q, src0, dst0)
            pltpu.async_copy(fts_h.at[src0], rows0, sem0).wait()
            scat(rows0, dst0)

        plsc.subcore_barrier()
        pltpu.sync_copy(acc.at[pl.ds(sid * RPS, RPS)],
                        out_h.at[pl.ds(cid * N + sid * RPS, RPS)])

        @pl.when(sid == 0)
        def _():
            pltpu.sync_copy(acc.at[pl.ds(NS * RPS, TAIL)],
                            out_h.at[pl.ds(cid * N + NS * RPS, TAIL)])

    return k(fts2, srcx, dst, zrows)


def _post_body(agg_ref, b_ref, a_ref, m_ref, sb_ref, h_ref, ret_ref, cm_ref):
    g = pl.program_id(0)
    x = agg_ref[0] + b_ref[...]
    a = a_ref[0, 0]
    h = jnp.where(x > 0.0, x, a * x)

    @pl.when(g == 0)
    def _():
        h_ref[0] = h
        c = jax.nn.sigmoid(jnp.sum(h, axis=0, keepdims=True) * (1.0 / N))
        cm_ref[...] = jnp.dot(c, m_ref[...], preferred_element_type=jnp.float32)

    s = jnp.sum(h * cm_ref[...], axis=-1)
    ret_ref[...] = s.reshape(1, 1, N) + sb_ref[...]


def _post(agg, b2, a2, M, sb):
    return pl.pallas_call(
        _post_body,
        grid=(2,),
        in_specs=[pl.BlockSpec((1, N, HID), lambda g: (g, 0, 0)),
                  pl.BlockSpec((1, HID), lambda g: (0, 0)),
                  pl.BlockSpec((1, 1), lambda g: (0, 0)),
                  pl.BlockSpec((HID, HID), lambda g: (0, 0)),
                  pl.BlockSpec((1, 1, N), lambda g: (g, 0, 0))],
        out_specs=[pl.BlockSpec((1, N, HID), lambda g: (0, 0, 0)),
                   pl.BlockSpec((1, 1, N), lambda g: (g, 0, 0))],
        out_shape=[jax.ShapeDtypeStruct((1, N, HID), jnp.float32),
                   jax.ShapeDtypeStruct((2, 1, N), jnp.float32)],
        scratch_shapes=[pltpu.VMEM((1, HID), jnp.float32)],
    )(agg, b2, a2, M, sb)


def kernel(seq1, seq2, adj, sparse, msk, samp_bias1, samp_bias2, W, b_gcn,
           prelu_a, M, disc_bias):
    seqs = jnp.concatenate([seq1, seq2], axis=0)
    fts2 = _mm(seqs, W).reshape(2 * N, HID)
    src = adj[0]
    dst = adj[1]
    srcx = jnp.concatenate([src, src + N], axis=0)
    zrows = jnp.zeros((RPS, HID), jnp.float32)
    agg = _sc_agg(fts2, srcx, dst, zrows).reshape(2, N, HID)
    b2 = b_gcn.reshape(1, HID)
    a2 = prelu_a.reshape(1, 1)
    sb = (jnp.stack([samp_bias1[0], samp_bias2[0]])
          .reshape(2, 1, N) + disc_bias)
    h1, ret2 = _post(agg, b2, a2, M, sb)
    ret = ret2.reshape(1, 2 * N)
    return (ret, h1)


# speedup vs baseline: 11.3562x; 1.1926x over previous
import functools

import jax
import jax.numpy as jnp
from jax import lax
from jax.experimental import pallas as pl
from jax.experimental.pallas import tpu as pltpu
from jax.experimental.pallas import tpu_sc as plsc

N = 10000
E = 320000
FT = 128
HID = 128
NS = 16
CK = 128
NCHT = E // CK
CPS = NCHT // NS
XTRA = NCHT - CPS * NS
EPS = CPS * CK
SB = 52
NSB = CPS // SB
RPS = 624
TAIL = N - NS * RPS


def _mm_body(s_ref, w_ref, o_ref):
    o_ref[0] = jnp.dot(s_ref[0], w_ref[...], preferred_element_type=jnp.float32)


def _mm(seqs, W):
    BR = 2000
    return pl.pallas_call(
        _mm_body,
        grid=(2, N // BR),
        in_specs=[pl.BlockSpec((1, BR, FT), lambda g, i: (g, i, 0)),
                  pl.BlockSpec((FT, HID), lambda g, i: (0, 0))],
        out_specs=pl.BlockSpec((1, BR, HID), lambda g, i: (g, i, 0)),
        out_shape=jax.ShapeDtypeStruct((2, N, HID), jnp.float32),
    )(seqs, W)


def _sc_agg(fts2, srcx, dstp, zrows):
    @functools.partial(
        pl.kernel,
        out_type=jax.ShapeDtypeStruct((2 * N, HID), jnp.float32),
        mesh=plsc.VectorSubcoreMesh(core_axis_name="c", subcore_axis_name="s"),
        scratch_types=[
            pltpu.VMEM((SB * CK,), jnp.int32),
            pltpu.VMEM((SB, CK), jnp.int32),
            pltpu.VMEM((CK, HID), jnp.float32),
            pltpu.VMEM((CK, HID), jnp.float32),
            pltpu.SemaphoreType.DMA,
            pltpu.SemaphoreType.DMA,
            pltpu.VMEM_SHARED((N, HID), jnp.float32),
        ],
    )
    def k(fts_h, srcx_h, dstp_h, z_h, out_h, src_sb, dst_sb,
          rows0, rows1, sem0, sem1, acc):
        cid = lax.axis_index("c")
        sid = lax.axis_index("s")
        rows = (rows0, rows1)
        sems = (sem0, sem1)

        pltpu.sync_copy(z_h, acc.at[pl.ds(sid * RPS, RPS)])

        @pl.when(sid == 0)
        def _():
            pltpu.sync_copy(z_h.at[pl.ds(0, TAIL)],
                            acc.at[pl.ds(NS * RPS, TAIL)])

        plsc.subcore_barrier()

        def issue(j, rv, sem):
            pltpu.async_copy(fts_h.at[src_sb.at[pl.ds(j * CK, CK)]], rv, sem)

        def wait(rv, sem):
            pltpu.make_async_copy(fts_h.at[pl.ds(0, CK)], rv, sem).wait()

        def scat(rv, j):
            pltpu.sync_copy(rv, acc.at[dst_sb.at[j]], add=True)

        def run_sb(s, nchunks):
            pltpu.sync_copy(
                srcx_h.at[pl.ds(cid * E + sid * EPS + s * SB * CK, SB * CK)],
                src_sb)
            pltpu.sync_copy(dstp_h.at[sid, s], dst_sb)
            issue(0, rows[0], sems[0])
            if nchunks > 1:
                issue(1, rows[1], sems[1])
            for j in range(nchunks):
                p = j % 2
                wait(rows[p], sems[p])
                scat(rows[p], j)
                if j + 2 < nchunks:
                    issue(j + 2, rows[p], sems[p])

        def body(s, carry):
            run_sb(s, SB)
            return carry

        lax.fori_loop(0, NSB, body, 0)

        @pl.when(sid < XTRA)
        def _():
            pltpu.sync_copy(
                srcx_h.at[pl.ds(cid * E + (NS * CPS + sid) * CK, CK)],
                src_sb.at[pl.ds(0, CK)])
            pltpu.sync_copy(dstp_h.at[sid, NSB], dst_sb)
            issue(0, rows[0], sems[0])
            wait(rows[0], sems[0])
            scat(rows[0], 0)

        plsc.subcore_barrier()
        pltpu.sync_copy(acc.at[pl.ds(sid * RPS, RPS)],
                        out_h.at[pl.ds(cid * N + sid * RPS, RPS)])

        @pl.when(sid == 0)
        def _():
            pltpu.sync_copy(acc.at[pl.ds(NS * RPS, TAIL)],
                            out_h.at[pl.ds(cid * N + NS * RPS, TAIL)])

    return k(fts2, srcx, dstp, zrows)


def _post_body(agg_ref, b_ref, a_ref, m_ref, sb_ref, h_ref, ret_ref, cm_ref):
    g = pl.program_id(0)
    x = agg_ref[0] + b_ref[...]
    a = a_ref[0, 0]
    h = jnp.where(x > 0.0, x, a * x)

    @pl.when(g == 0)
    def _():
        h_ref[0] = h
        c = jax.nn.sigmoid(jnp.sum(h, axis=0, keepdims=True) * (1.0 / N))
        cm_ref[...] = jnp.dot(c, m_ref[...], preferred_element_type=jnp.float32)

    s = jnp.sum(h * cm_ref[...], axis=-1)
    ret_ref[...] = s.reshape(1, 1, N) + sb_ref[...]


def _post(agg, b2, a2, M, sb):
    return pl.pallas_call(
        _post_body,
        grid=(2,),
        in_specs=[pl.BlockSpec((1, N, HID), lambda g: (g, 0, 0)),
                  pl.BlockSpec((1, HID), lambda g: (0, 0)),
                  pl.BlockSpec((1, 1), lambda g: (0, 0)),
                  pl.BlockSpec((HID, HID), lambda g: (0, 0)),
                  pl.BlockSpec((1, 1, N), lambda g: (g, 0, 0))],
        out_specs=[pl.BlockSpec((1, N, HID), lambda g: (0, 0, 0)),
                   pl.BlockSpec((1, 1, N), lambda g: (g, 0, 0))],
        out_shape=[jax.ShapeDtypeStruct((1, N, HID), jnp.float32),
                   jax.ShapeDtypeStruct((2, 1, N), jnp.float32)],
        scratch_shapes=[pltpu.VMEM((1, HID), jnp.float32)],
    )(agg, b2, a2, M, sb)


def kernel(seq1, seq2, adj, sparse, msk, samp_bias1, samp_bias2, W, b_gcn,
           prelu_a, M, disc_bias):
    seqs = jnp.concatenate([seq1, seq2], axis=0)
    fts2 = _mm(seqs, W).reshape(2 * N, HID)
    src = adj[0]
    dst = adj[1]
    srcx = jnp.concatenate([src, src + N], axis=0)
    dst_main = dst[:NS * EPS].reshape(NS, NSB, SB, CK)
    dst_x = dst[NS * EPS:].reshape(XTRA, CK)
    dstp = jnp.concatenate(
        [dst_main, jnp.zeros((NS, 1, SB, CK), jnp.int32)], axis=1)
    dstp = dstp.at[:XTRA, NSB, 0].set(dst_x)
    zrows = jnp.zeros((RPS, HID), jnp.float32)
    agg = _sc_agg(fts2, srcx, dstp, zrows).reshape(2, N, HID)
    b2 = b_gcn.reshape(1, HID)
    a2 = prelu_a.reshape(1, 1)
    sb = (jnp.stack([samp_bias1[0], samp_bias2[0]])
          .reshape(2, 1, N) + disc_bias)
    h1, ret2 = _post(agg, b2, a2, M, sb)
    ret = ret2.reshape(1, 2 * N)
    return (ret, h1)


# speedup vs baseline: 11.8497x; 1.0435x over previous
import functools

import jax
import jax.numpy as jnp
from jax import lax
from jax.experimental import pallas as pl
from jax.experimental.pallas import tpu as pltpu
from jax.experimental.pallas import tpu_sc as plsc

N = 10000
E = 320000
FT = 128
HID = 128
NS = 16
CK = 80
EPS = E // NS
CHN = EPS // CK
DR = 4
DI = 6
UN = 12
TRIP = (CHN + 2 + UN - 1) // UN
RPS = 624
TAIL = N - NS * RPS


def _mm_body(s_ref, w_ref, o_ref):
    o_ref[0] = jnp.dot(s_ref[0], w_ref[...], preferred_element_type=jnp.float32)


def _mm(seqs, W):
    BR = 2000
    return pl.pallas_call(
        _mm_body,
        grid=(2, N // BR),
        in_specs=[pl.BlockSpec((1, BR, FT), lambda g, i: (g, i, 0)),
                  pl.BlockSpec((FT, HID), lambda g, i: (0, 0))],
        out_specs=pl.BlockSpec((1, BR, HID), lambda g, i: (g, i, 0)),
        out_shape=jax.ShapeDtypeStruct((2, N, HID), jnp.float32),
    )(seqs, W)


def _sc_agg(fts2, srcx, dsth, zrows):
    @functools.partial(
        pl.kernel,
        out_type=jax.ShapeDtypeStruct((2 * N, HID), jnp.float32),
        mesh=plsc.VectorSubcoreMesh(core_axis_name="c", subcore_axis_name="s"),
        scratch_types=(
            [pltpu.VMEM((CK,), jnp.int32) for _ in range(DI)]
            + [pltpu.VMEM((CK,), jnp.int32) for _ in range(DI)]
            + [pltpu.VMEM((CK, HID), jnp.float32) for _ in range(DR)]
            + [pltpu.SemaphoreType.DMA] * (DI + 2 * DR)
            + [pltpu.VMEM_SHARED((N, HID), jnp.float32)]
        ),
    )
    def k(fts_h, srcx_h, dst_h, z_h, out_h, *refs):
        src_i = refs[0:DI]
        dst_i = refs[DI:2 * DI]
        rows = refs[2 * DI:2 * DI + DR]
        sem_i = refs[2 * DI + DR:2 * DI + DR + DI]
        sem_g = refs[2 * DI + DR + DI:2 * DI + DR + DI + DR]
        sem_s = refs[2 * DI + DR + DI + DR:2 * DI + DR + DI + 2 * DR]
        acc = refs[-1]
        cid = lax.axis_index("c")
        sid = lax.axis_index("s")

        pltpu.sync_copy(z_h, acc.at[pl.ds(sid * RPS, RPS)])

        @pl.when(sid == 0)
        def _():
            pltpu.sync_copy(z_h.at[pl.ds(0, TAIL)],
                            acc.at[pl.ds(NS * RPS, TAIL)])

        plsc.subcore_barrier()

        sbase = cid * E + sid * EPS
        dbase = sid * EPS

        def idx_load(q, u):
            pltpu.async_copy(srcx_h.at[pl.ds(sbase + q * CK, CK)],
                             src_i[u], sem_i[u])
            pltpu.async_copy(dst_h.at[pl.ds(dbase + q * CK, CK)],
                             dst_i[u], sem_i[u])

        def idx_wait(u):
            pltpu.make_async_copy(srcx_h.at[pl.ds(0, CK)],
                                  src_i[u], sem_i[u]).wait()
            pltpu.make_async_copy(dst_h.at[pl.ds(0, CK)],
                                  dst_i[u], sem_i[u]).wait()

        def g_issue(u, p):
            pltpu.async_copy(fts_h.at[src_i[u]], rows[p], sem_g[p])

        def g_wait(p):
            pltpu.make_async_copy(fts_h.at[pl.ds(0, CK)],
                                  rows[p], sem_g[p]).wait()

        def s_issue(p, u):
            pltpu.async_copy(rows[p], acc.at[dst_i[u]], sem_s[p], add=True)

        def s_wait(p):
            pltpu.make_async_copy(fts_h.at[pl.ds(0, CK)],
                                  rows[p], sem_s[p]).wait()

        for q in range(DR):
            idx_load(q, q)
        for q in range(2):
            idx_wait(q)
            g_issue(q, q)

        def body(t, carry):
            j0 = t * UN
            for kk in range(UN):
                j = j0 + kk

                @pl.when(j < CHN)
                def _(p=kk % DR, u=kk % DI):
                    g_wait(p)
                    s_issue(p, u)

                @pl.when((j >= 2) & (j < CHN + 2))
                def _(p=(kk + 2) % DR):
                    s_wait(p)

                @pl.when(j < CHN - 2)
                def _(p=(kk + 2) % DR, u=(kk + 2) % DI):
                    idx_wait(u)
                    g_issue(u, p)

                @pl.when(j < CHN - 4)
                def _(u=(kk + 4) % DI):
                    idx_load(j + 4, u)

            return carry

        lax.fori_loop(0, TRIP, body, 0)
        plsc.subcore_barrier()
        pltpu.sync_copy(acc.at[pl.ds(sid * RPS, RPS)],
                        out_h.at[pl.ds(cid * N + sid * RPS, RPS)])

        @pl.when(sid == 0)
        def _():
            pltpu.sync_copy(acc.at[pl.ds(NS * RPS, TAIL)],
                            out_h.at[pl.ds(cid * N + NS * RPS, TAIL)])

    return k(fts2, srcx, dsth, zrows)


def _post_body(agg_ref, b_ref, a_ref, m_ref, sb_ref, h_ref, ret_ref, cm_ref):
    g = pl.program_id(0)
    x = agg_ref[0] + b_ref[...]
    a = a_ref[0, 0]
    h = jnp.where(x > 0.0, x, a * x)

    @pl.when(g == 0)
    def _():
        h_ref[0] = h
        c = jax.nn.sigmoid(jnp.sum(h, axis=0, keepdims=True) * (1.0 / N))
        cm_ref[...] = jnp.dot(c, m_ref[...], preferred_element_type=jnp.float32)

    s = jnp.sum(h * cm_ref[...], axis=-1)
    ret_ref[...] = s.reshape(1, 1, N) + sb_ref[...]


def _post(agg, b2, a2, M, sb):
    return pl.pallas_call(
        _post_body,
        grid=(2,),
        in_specs=[pl.BlockSpec((1, N, HID), lambda g: (g, 0, 0)),
                  pl.BlockSpec((1, HID), lambda g: (0, 0)),
                  pl.BlockSpec((1, 1), lambda g: (0, 0)),
                  pl.BlockSpec((HID, HID), lambda g: (0, 0)),
                  pl.BlockSpec((1, 1, N), lambda g: (g, 0, 0))],
        out_specs=[pl.BlockSpec((1, N, HID), lambda g: (0, 0, 0)),
                   pl.BlockSpec((1, 1, N), lambda g: (g, 0, 0))],
        out_shape=[jax.ShapeDtypeStruct((1, N, HID), jnp.float32),
                   jax.ShapeDtypeStruct((2, 1, N), jnp.float32)],
        scratch_shapes=[pltpu.VMEM((1, HID), jnp.float32)],
    )(agg, b2, a2, M, sb)


def kernel(seq1, seq2, adj, sparse, msk, samp_bias1, samp_bias2, W, b_gcn,
           prelu_a, M, disc_bias):
    seqs = jnp.concatenate([seq1, seq2], axis=0)
    fts2 = _mm(seqs, W).reshape(2 * N, HID)
    src = adj[0]
    dst = adj[1]
    srcx = jnp.concatenate([src, src + N], axis=0)
    zrows = jnp.zeros((RPS, HID), jnp.float32)
    agg = _sc_agg(fts2, srcx, dst, zrows).reshape(2, N, HID)
    b2 = b_gcn.reshape(1, HID)
    a2 = prelu_a.reshape(1, 1)
    sb = (jnp.stack([samp_bias1[0], samp_bias2[0]])
          .reshape(2, 1, N) + disc_bias)
    h1, ret2 = _post(agg, b2, a2, M, sb)
    ret = ret2.reshape(1, 2 * N)
    return (ret, h1)


# speedup vs baseline: 12.5867x; 1.0622x over previous
import functools

import jax
import jax.numpy as jnp
from jax import lax
from jax.experimental import pallas as pl
from jax.experimental.pallas import tpu as pltpu
from jax.experimental.pallas import tpu_sc as plsc

N = 10000
E = 320000
FT = 128
HID = 128
NS = 16
CK = 128
NCHT = E // CK
CPS = NCHT // NS
XTRA = NCHT - CPS * NS
DR = 3
DI = 6
UN = 6
TRIP = (CPS + 1 + 2 + UN - 1) // UN
RPS = 624
TAIL = N - NS * RPS


def _mm_body(s_ref, w_ref, o_ref):
    o_ref[0] = jnp.dot(s_ref[0], w_ref[...], preferred_element_type=jnp.float32)


def _mm(seqs, W):
    BR = 2000
    return pl.pallas_call(
        _mm_body,
        grid=(2, N // BR),
        in_specs=[pl.BlockSpec((1, BR, FT), lambda g, i: (g, i, 0)),
                  pl.BlockSpec((FT, HID), lambda g, i: (0, 0))],
        out_specs=pl.BlockSpec((1, BR, HID), lambda g, i: (g, i, 0)),
        out_shape=jax.ShapeDtypeStruct((2, N, HID), jnp.float32),
    )(seqs, W)


def _sc_agg(fts2, sdp, zrows):
    @functools.partial(
        pl.kernel,
        out_type=jax.ShapeDtypeStruct((2 * N, HID), jnp.float32),
        mesh=plsc.VectorSubcoreMesh(core_axis_name="c", subcore_axis_name="s"),
        scratch_types=(
            [pltpu.VMEM((2, CK), jnp.int32) for _ in range(DI)]
            + [pltpu.VMEM((CK, HID), jnp.float32) for _ in range(DR)]
            + [pltpu.SemaphoreType.DMA] * (DI + 2 * DR)
            + [pltpu.VMEM_SHARED((N, HID), jnp.float32)]
        ),
    )
    def k(fts_h, sd_h, z_h, out_h, *refs):
        sdi = refs[0:DI]
        rows = refs[DI:DI + DR]
        sem_i = refs[DI + DR:2 * DI + DR]
        sem_g = refs[2 * DI + DR:2 * DI + 2 * DR]
        sem_s = refs[2 * DI + 2 * DR:2 * DI + 3 * DR]
        acc = refs[-1]
        cid = lax.axis_index("c")
        sid = lax.axis_index("s")

        pltpu.sync_copy(z_h, acc.at[pl.ds(sid * RPS, RPS)])

        @pl.when(sid == 0)
        def _():
            pltpu.sync_copy(z_h.at[pl.ds(0, TAIL)],
                            acc.at[pl.ds(NS * RPS, TAIL)])

        plsc.subcore_barrier()

        ne = CPS + jnp.where(sid < XTRA, 1, 0)

        def chunk_id(j):
            return jnp.where(j < CPS, sid * CPS + j, NCHT - XTRA + sid)

        def idx_load(j, u):
            pltpu.async_copy(sd_h.at[cid, chunk_id(j)], sdi[u], sem_i[u])

        def idx_wait(u):
            pltpu.make_async_copy(sd_h.at[0, 0], sdi[u], sem_i[u]).wait()

        def g_issue(u, p):
            pltpu.async_copy(fts_h.at[sdi[u].at[0]], rows[p], sem_g[p])

        def g_wait(p):
            pltpu.make_async_copy(fts_h.at[pl.ds(0, CK)],
                                  rows[p], sem_g[p]).wait()

        def s_issue(p, u):
            pltpu.async_copy(rows[p], acc.at[sdi[u].at[1]], sem_s[p], add=True)

        def s_wait(p):
            pltpu.make_async_copy(fts_h.at[pl.ds(0, CK)],
                                  rows[p], sem_s[p]).wait()

        for q in range(4):
            idx_load(q, q)
        for q in range(2):
            idx_wait(q)
            g_issue(q, q)

        def body(t, carry):
            j0 = t * UN
            for kk in range(UN):
                j = j0 + kk

                @pl.when(j < ne)
                def _(p=kk % DR, u=kk % DI):
                    g_wait(p)
                    s_issue(p, u)

                @pl.when((j >= 1) & (j < ne + 1))
                def _(p=(kk + 2) % DR):
                    s_wait(p)

                @pl.when(j < ne - 2)
                def _(p=(kk + 2) % DR, u=(kk + 2) % DI):
                    idx_wait(u)
                    g_issue(u, p)

                @pl.when(j < ne - 4)
                def _(u=(kk + 4) % DI):
                    idx_load(j + 4, u)

            return carry

        lax.fori_loop(0, TRIP, body, 0)
        plsc.subcore_barrier()
        pltpu.sync_copy(acc.at[pl.ds(sid * RPS, RPS)],
                        out_h.at[pl.ds(cid * N + sid * RPS, RPS)])

        @pl.when(sid == 0)
        def _():
            pltpu.sync_copy(acc.at[pl.ds(NS * RPS, TAIL)],
                            out_h.at[pl.ds(cid * N + NS * RPS, TAIL)])

    return k(fts2, sdp, zrows)


def _post_body(agg_ref, b_ref, a_ref, m_ref, sb_ref, h_ref, ret_ref, cm_ref):
    g = pl.program_id(0)
    x = agg_ref[0] + b_ref[...]
    a = a_ref[0, 0]
    h = jnp.where(x > 0.0, x, a * x)

    @pl.when(g == 0)
    def _():
        h_ref[0] = h
        c = jax.nn.sigmoid(jnp.sum(h, axis=0, keepdims=True) * (1.0 / N))
        cm_ref[...] = jnp.dot(c, m_ref[...], preferred_element_type=jnp.float32)

    s = jnp.sum(h * cm_ref[...], axis=-1)
    ret_ref[...] = s.reshape(1, 1, N) + sb_ref[...]


def _post(agg, b2, a2, M, sb):
    return pl.pallas_call(
        _post_body,
        grid=(2,),
        in_specs=[pl.BlockSpec((1, N, HID), lambda g: (g, 0, 0)),
                  pl.BlockSpec((1, HID), lambda g: (0, 0)),
                  pl.BlockSpec((1, 1), lambda g: (0, 0)),
                  pl.BlockSpec((HID, HID), lambda g: (0, 0)),
                  pl.BlockSpec((1, 1, N), lambda g: (g, 0, 0))],
        out_specs=[pl.BlockSpec((1, N, HID), lambda g: (0, 0, 0)),
                   pl.BlockSpec((1, 1, N), lambda g: (g, 0, 0))],
        out_shape=[jax.ShapeDtypeStruct((1, N, HID), jnp.float32),
                   jax.ShapeDtypeStruct((2, 1, N), jnp.float32)],
        scratch_shapes=[pltpu.VMEM((1, HID), jnp.float32)],
    )(agg, b2, a2, M, sb)


def kernel(seq1, seq2, adj, sparse, msk, samp_bias1, samp_bias2, W, b_gcn,
           prelu_a, M, disc_bias):
    seqs = jnp.concatenate([seq1, seq2], axis=0)
    fts2 = _mm(seqs, W).reshape(2 * N, HID)
    src = adj[0]
    dst = adj[1]
    srcs = jnp.stack([src, src + N]).reshape(2, NCHT, CK)
    dsts = jnp.broadcast_to(dst.reshape(1, NCHT, CK), (2, NCHT, CK))
    sdp = jnp.stack([srcs, dsts], axis=2)
    zrows = jnp.zeros((RPS, HID), jnp.float32)
    agg = _sc_agg(fts2, sdp, zrows).reshape(2, N, HID)
    b2 = b_gcn.reshape(1, HID)
    a2 = prelu_a.reshape(1, 1)
    sb = (jnp.stack([samp_bias1[0], samp_bias2[0]])
          .reshape(2, 1, N) + disc_bias)
    h1, ret2 = _post(agg, b2, a2, M, sb)
    ret = ret2.reshape(1, 2 * N)
    return (ret, h1)


# speedup vs baseline: 14.8221x; 1.1776x over previous
import functools

import jax
import jax.numpy as jnp
from jax import lax
from jax.experimental import pallas as pl
from jax.experimental.pallas import tpu as pltpu
from jax.experimental.pallas import tpu_sc as plsc

N = 10000
E = 320000
FT = 128
HID = 128
NS = 16
CK = 128
NCHT = E // CK
CPS = NCHT // NS
XTRA = NCHT - CPS * NS
DR = 3
DI = 6
UN = 6
TRIP = (CPS + 1 + 2 + UN - 1) // UN
RPS = 624
TAIL = N - NS * RPS


def _mm_body(s1_ref, s2_ref, w_ref, o_ref):
    o_ref[0] = jnp.dot(s1_ref[0], w_ref[...], preferred_element_type=jnp.float32)
    o_ref[1] = jnp.dot(s2_ref[0], w_ref[...], preferred_element_type=jnp.float32)


def _mm(seq1, seq2, W):
    BR = 2000
    return pl.pallas_call(
        _mm_body,
        grid=(N // BR,),
        in_specs=[pl.BlockSpec((1, BR, FT), lambda i: (0, i, 0)),
                  pl.BlockSpec((1, BR, FT), lambda i: (0, i, 0)),
                  pl.BlockSpec((FT, HID), lambda i: (0, 0))],
        out_specs=pl.BlockSpec((2, BR, HID), lambda i: (0, i, 0)),
        out_shape=jax.ShapeDtypeStruct((2, N, HID), jnp.float32),
    )(seq1, seq2, W)


def _sc_agg(fts2, adj2, zrows):
    @functools.partial(
        pl.kernel,
        out_type=jax.ShapeDtypeStruct((2 * N, HID), jnp.float32),
        mesh=plsc.VectorSubcoreMesh(core_axis_name="c", subcore_axis_name="s"),
        scratch_types=(
            [pltpu.VMEM((CK,), jnp.int32) for _ in range(DI)]
            + [pltpu.VMEM((CK,), jnp.int32) for _ in range(DI)]
            + [pltpu.VMEM((CK, HID), jnp.float32) for _ in range(DR)]
            + [pltpu.SemaphoreType.DMA] * (DI + 2 * DR)
            + [pltpu.VMEM_SHARED((N, HID), jnp.float32)]
        ),
    )
    def k(fts_h, adj_h, z_h, out_h, *refs):
        src_i = refs[0:DI]
        dst_i = refs[DI:2 * DI]
        rows = refs[2 * DI:2 * DI + DR]
        sem_i = refs[2 * DI + DR:3 * DI + DR]
        sem_g = refs[3 * DI + DR:3 * DI + 2 * DR]
        sem_s = refs[3 * DI + 2 * DR:3 * DI + 3 * DR]
        acc = refs[-1]
        cid = lax.axis_index("c")
        sid = lax.axis_index("s")
        ftsg = fts_h.at[pl.ds(cid * N, N)]

        pltpu.sync_copy(z_h, acc.at[pl.ds(sid * RPS, RPS)])

        @pl.when(sid == 0)
        def _():
            pltpu.sync_copy(z_h.at[pl.ds(0, TAIL)],
                            acc.at[pl.ds(NS * RPS, TAIL)])

        plsc.subcore_barrier()

        ne = CPS + jnp.where(sid < XTRA, 1, 0)

        def chunk_id(j):
            return jnp.where(j < CPS, sid * CPS + j, NCHT - XTRA + sid)

        def idx_load(j, u):
            q = chunk_id(j)
            pltpu.async_copy(adj_h.at[pl.ds(q * CK, CK)], src_i[u], sem_i[u])
            pltpu.async_copy(adj_h.at[pl.ds(E + q * CK, CK)],
                             dst_i[u], sem_i[u])

        def idx_wait(u):
            pltpu.make_async_copy(adj_h.at[pl.ds(0, CK)],
                                  src_i[u], sem_i[u]).wait()
            pltpu.make_async_copy(adj_h.at[pl.ds(0, CK)],
                                  dst_i[u], sem_i[u]).wait()

        def g_issue(u, p):
            pltpu.async_copy(ftsg.at[src_i[u]], rows[p], sem_g[p])

        def g_wait(p):
            pltpu.make_async_copy(fts_h.at[pl.ds(0, CK)],
                                  rows[p], sem_g[p]).wait()

        def s_issue(p, u):
            pltpu.async_copy(rows[p], acc.at[dst_i[u]], sem_s[p], add=True)

        def s_wait(p):
            pltpu.make_async_copy(fts_h.at[pl.ds(0, CK)],
                                  rows[p], sem_s[p]).wait()

        for q in range(4):
            idx_load(q, q)
        for q in range(2):
            idx_wait(q)
            g_issue(q, q)

        def body(t, carry):
            j0 = t * UN
            for kk in range(UN):
                j = j0 + kk

                @pl.when(j < ne)
                def _(p=kk % DR, u=kk % DI):
                    g_wait(p)
                    s_issue(p, u)

                @pl.when((j >= 1) & (j < ne + 1))
                def _(p=(kk + 2) % DR):
                    s_wait(p)

                @pl.when(j < ne - 2)
                def _(p=(kk + 2) % DR, u=(kk + 2) % DI):
                    idx_wait(u)
                    g_issue(u, p)

                @pl.when(j < ne - 4)
                def _(u=(kk + 4) % DI):
                    idx_load(j + 4, u)

            return carry

        lax.fori_loop(0, TRIP, body, 0)
        plsc.subcore_barrier()
        pltpu.sync_copy(acc.at[pl.ds(sid * RPS, RPS)],
                        out_h.at[pl.ds(cid * N + sid * RPS, RPS)])

        @pl.when(sid == 0)
        def _():
            pltpu.sync_copy(acc.at[pl.ds(NS * RPS, TAIL)],
                            out_h.at[pl.ds(cid * N + NS * RPS, TAIL)])

    return k(fts2, adj2, zrows)


def _post_body(agg_ref, b_ref, a_ref, m_ref, sb_ref, h_ref, ret_ref, cm_ref):
    g = pl.program_id(0)
    x = agg_ref[0] + b_ref[...]
    a = a_ref[0, 0]
    h = jnp.where(x > 0.0, x, a * x)

    @pl.when(g == 0)
    def _():
        h_ref[0] = h
        c = jax.nn.sigmoid(jnp.sum(h, axis=0, keepdims=True) * (1.0 / N))
        cm_ref[...] = jnp.dot(c, m_ref[...], preferred_element_type=jnp.float32)

    s = lax.dot_general(cm_ref[...], h, (((1,), (1,)), ((), ())),
                        preferred_element_type=jnp.float32)
    ret_ref[...] = s[None] + sb_ref[...]


def _post(agg, b2, a2, M, sb):
    return pl.pallas_call(
        _post_body,
        grid=(2,),
        in_specs=[pl.BlockSpec((1, N, HID), lambda g: (g, 0, 0)),
                  pl.BlockSpec((1, HID), lambda g: (0, 0)),
                  pl.BlockSpec((1, 1), lambda g: (0, 0)),
                  pl.BlockSpec((HID, HID), lambda g: (0, 0)),
                  pl.BlockSpec((1, 1, N), lambda g: (g, 0, 0))],
        out_specs=[pl.BlockSpec((1, N, HID), lambda g: (0, 0, 0)),
                   pl.BlockSpec((1, 1, N), lambda g: (g, 0, 0))],
        out_shape=[jax.ShapeDtypeStruct((1, N, HID), jnp.float32),
                   jax.ShapeDtypeStruct((2, 1, N), jnp.float32)],
        scratch_shapes=[pltpu.VMEM((1, HID), jnp.float32)],
    )(agg, b2, a2, M, sb)


def kernel(seq1, seq2, adj, sparse, msk, samp_bias1, samp_bias2, W, b_gcn,
           prelu_a, M, disc_bias):
    fts2 = _mm(seq1, seq2, W).reshape(2 * N, HID)
    zrows = jnp.zeros((RPS, HID), jnp.float32)
    agg = _sc_agg(fts2, adj.reshape(2 * E), zrows).reshape(2, N, HID)
    b2 = b_gcn.reshape(1, HID)
    a2 = prelu_a.reshape(1, 1)
    sb = (jnp.stack([samp_bias1[0], samp_bias2[0]])
          .reshape(2, 1, N) + disc_bias)
    h1, ret2 = _post(agg, b2, a2, M, sb)
    ret = ret2.reshape(1, 2 * N)
    return (ret, h1)
